# baseline (device time: 211768 ns/iter reference)
import jax
import jax.numpy as jnp
from jax import lax
from jax.experimental import pallas as pl
from jax.experimental.pallas import tpu as pltpu

N_DEV = 16


def kernel(x, w_mat, scale_x, scale_w):
    m_per, k = x.shape
    _, n_per = w_mat.shape

    def body(x_ref, w_ref, sx_ref, sw_ref, out_ref, xg_ref, w8_ref,
             send_sems, recv_sems):
        my = lax.axis_index("i")
        left = lax.rem(my + N_DEV - 1, N_DEV)
        right = lax.rem(my + 1, N_DEV)

        barrier_sem = pltpu.get_barrier_semaphore()
        pl.semaphore_signal(barrier_sem, inc=1, device_id=(left,),
                            device_id_type=pl.DeviceIdType.MESH)
        pl.semaphore_signal(barrier_sem, inc=1, device_id=(right,),
                            device_id_type=pl.DeviceIdType.MESH)
        pl.semaphore_wait(barrier_sem, 2)

        w8_ref[...] = w_ref[...].astype(jnp.float8_e5m2)
        xg_ref[my] = x_ref[...].astype(jnp.float8_e5m2)

        scale = sx_ref[0] * sw_ref[0]

        def chunk_out(org):
            xc = xg_ref[org]
            acc = lax.dot_general(
                xc, w8_ref[...], (((1,), (0,)), ((), ())),
                preferred_element_type=jnp.float32)
            out_ref[pl.ds(org * m_per, m_per), :] = jnp.maximum(
                acc * scale, 0.0)

        chunk_out(my)

        for s in range(N_DEV - 1):
            o_send = lax.rem(my - s + N_DEV, N_DEV)
            o_recv = lax.rem(my - s - 1 + N_DEV, N_DEV)
            send = pltpu.make_async_remote_copy(
                src_ref=xg_ref.at[o_send],
                dst_ref=xg_ref.at[o_send],
                send_sem=send_sems.at[o_send],
                recv_sem=recv_sems.at[o_send],
                device_id=(right,),
                device_id_type=pl.DeviceIdType.MESH,
            )
            send.start()
            recv = pltpu.make_async_remote_copy(
                src_ref=xg_ref.at[o_recv],
                dst_ref=xg_ref.at[o_recv],
                send_sem=send_sems.at[o_recv],
                recv_sem=recv_sems.at[o_recv],
                device_id=(right,),
                device_id_type=pl.DeviceIdType.MESH,
            )
            recv.wait_recv()
            send.wait_send()
            chunk_out(o_recv)

    return pl.pallas_call(
        body,
        out_shape=jax.ShapeDtypeStruct((N_DEV * m_per, n_per), jnp.float32),
        in_specs=[
            pl.BlockSpec(memory_space=pltpu.VMEM),
            pl.BlockSpec(memory_space=pltpu.VMEM),
            pl.BlockSpec(memory_space=pltpu.SMEM),
            pl.BlockSpec(memory_space=pltpu.SMEM),
        ],
        out_specs=pl.BlockSpec(memory_space=pltpu.VMEM),
        scratch_shapes=[
            pltpu.VMEM((N_DEV, m_per, k), jnp.float8_e5m2),
            pltpu.VMEM((k, n_per), jnp.float8_e5m2),
            pltpu.SemaphoreType.DMA((N_DEV,)),
            pltpu.SemaphoreType.DMA((N_DEV,)),
        ],
        compiler_params=pltpu.CompilerParams(collective_id=0),
    )(x, w_mat, scale_x, scale_w)


# device time: 114833 ns/iter; 1.8441x vs baseline; 1.8441x over previous
import jax
import jax.numpy as jnp
from jax import lax
from jax.experimental import pallas as pl
from jax.experimental.pallas import tpu as pltpu

N_DEV = 16
N_R = 8
N_L = 7


def kernel(x, w_mat, scale_x, scale_w):
    m_per, k = x.shape
    _, n_per = w_mat.shape

    def body(x_ref, w_ref, sx_ref, sw_ref, out_ref, xg_ref, w8_ref,
             send_sems_r, send_sems_l, recv_sems):
        my = lax.axis_index("i")
        left = lax.rem(my + N_DEV - 1, N_DEV)
        right = lax.rem(my + 1, N_DEV)

        barrier_sem = pltpu.get_barrier_semaphore()
        pl.semaphore_signal(barrier_sem, inc=1, device_id=(left,),
                            device_id_type=pl.DeviceIdType.MESH)
        pl.semaphore_signal(barrier_sem, inc=1, device_id=(right,),
                            device_id_type=pl.DeviceIdType.MESH)
        pl.semaphore_wait(barrier_sem, 2)

        xg_ref[my] = x_ref[...].astype(jnp.float8_e5m2)
        w8_ref[...] = w_ref[...].astype(jnp.float8_e5m2)

        scale = sx_ref[0] * sw_ref[0]

        def chunk_out(org):
            xc = xg_ref[org]
            acc = lax.dot_general(
                xc, w8_ref[...], (((1,), (0,)), ((), ())),
                preferred_element_type=jnp.float32)
            out_ref[pl.ds(org * m_per, m_per), :] = jnp.maximum(
                acc * scale, 0.0)

        def mk(org, dev, send_sem):
            return pltpu.make_async_remote_copy(
                src_ref=xg_ref.at[org],
                dst_ref=xg_ref.at[org],
                send_sem=send_sem,
                recv_sem=recv_sems.at[org],
                device_id=(dev,),
                device_id_type=pl.DeviceIdType.MESH,
            )

        sends = []

        def start(org, dev, send_sem):
            s = mk(org, dev, send_sem)
            s.start()
            sends.append(s)

        start(my, right, send_sems_r.at[0])
        start(my, left, send_sems_l.at[0])
        chunk_out(my)

        for r in range(N_R):
            o_r = lax.rem(my - 1 - r + N_DEV, N_DEV)
            mk(o_r, right, send_sems_r.at[0]).wait_recv()
            if r + 1 < N_R:
                start(o_r, right, send_sems_r.at[r + 1])
            if r < N_L:
                o_l = lax.rem(my + 1 + r, N_DEV)
                mk(o_l, left, send_sems_l.at[0]).wait_recv()
                if r + 1 < N_L:
                    start(o_l, left, send_sems_l.at[r + 1])
            chunk_out(o_r)
            if r < N_L:
                chunk_out(o_l)

        for s in sends:
            s.wait_send()

    return pl.pallas_call(
        body,
        out_shape=jax.ShapeDtypeStruct((N_DEV * m_per, n_per), jnp.float32),
        in_specs=[
            pl.BlockSpec(memory_space=pltpu.VMEM),
            pl.BlockSpec(memory_space=pltpu.VMEM),
            pl.BlockSpec(memory_space=pltpu.SMEM),
            pl.BlockSpec(memory_space=pltpu.SMEM),
        ],
        out_specs=pl.BlockSpec(memory_space=pltpu.VMEM),
        scratch_shapes=[
            pltpu.VMEM((N_DEV, m_per, k), jnp.float8_e5m2),
            pltpu.VMEM((k, n_per), jnp.float8_e5m2),
            pltpu.SemaphoreType.DMA((N_R,)),
            pltpu.SemaphoreType.DMA((N_L,)),
            pltpu.SemaphoreType.DMA((N_DEV,)),
        ],
        compiler_params=pltpu.CompilerParams(collective_id=0),
    )(x, w_mat, scale_x, scale_w)


# device time: 110276 ns/iter; 1.9203x vs baseline; 1.0413x over previous
import jax
import jax.numpy as jnp
from jax import lax
from jax.experimental import pallas as pl
from jax.experimental.pallas import tpu as pltpu

N_DEV = 16
N_RND = 8
TOP, BOT = 0, 1


def kernel(x, w_mat, scale_x, scale_w):
    m_per, k = x.shape
    _, n_per = w_mat.shape
    m_half = m_per // 2

    def body(x_ref, w_ref, sx_ref, sw_ref, out_ref, xg_ref, w8_ref,
             send_r, send_l, recv_r, recv_l):
        my = lax.axis_index("i")
        left = lax.rem(my + N_DEV - 1, N_DEV)
        right = lax.rem(my + 1, N_DEV)

        barrier_sem = pltpu.get_barrier_semaphore()
        pl.semaphore_signal(barrier_sem, inc=1, device_id=(left,),
                            device_id_type=pl.DeviceIdType.MESH)
        pl.semaphore_signal(barrier_sem, inc=1, device_id=(right,),
                            device_id_type=pl.DeviceIdType.MESH)
        pl.semaphore_wait(barrier_sem, 2)

        xg_ref[my] = x_ref[...].astype(jnp.float8_e5m2)
        w8_ref[...] = w_ref[...].astype(jnp.float8_e5m2)

        scale = sx_ref[0] * sw_ref[0]

        def chunk_out(org):
            xc = xg_ref[org]
            acc = lax.dot_general(
                xc, w8_ref[...], (((1,), (0,)), ((), ())),
                preferred_element_type=jnp.float32)
            out_ref[pl.ds(org * m_per, m_per), :] = jnp.maximum(
                acc * scale, 0.0)

        def mk(org, dev, send_sem, recv_sem, half=None):
            if half is None:
                src = xg_ref.at[org]
            else:
                src = xg_ref.at[org, pl.ds(half * m_half, m_half)]
            return pltpu.make_async_remote_copy(
                src_ref=src, dst_ref=src,
                send_sem=send_sem, recv_sem=recv_sem,
                device_id=(dev,), device_id_type=pl.DeviceIdType.MESH,
            )

        sends = []

        def start(org, dev, send_sems, recv_sems, r, half=None):
            s = mk(org, dev, send_sems.at[r], recv_sems.at[r], half)
            s.start()
            sends.append(s)

        start(my, right, send_r, recv_r, 0)
        start(my, left, send_l, recv_l, 0)
        chunk_out(my)

        for r in range(N_RND - 1):
            o_r = lax.rem(my - 1 - r + N_DEV, N_DEV)
            o_l = lax.rem(my + 1 + r, N_DEV)
            mk(o_r, right, send_r.at[r], recv_r.at[r]).wait_recv()
            if r < N_RND - 2:
                start(o_r, right, send_r, recv_r, r + 1)
            else:
                start(o_r, right, send_r, recv_r, r + 1, half=TOP)
            mk(o_l, left, send_l.at[r], recv_l.at[r]).wait_recv()
            if r < N_RND - 2:
                start(o_l, left, send_l, recv_l, r + 1)
            else:
                start(o_l, left, send_l, recv_l, r + 1, half=BOT)
            chunk_out(o_r)
            chunk_out(o_l)

        o8 = lax.rem(my + N_DEV // 2, N_DEV)
        mk(o8, right, send_r.at[N_RND - 1], recv_r.at[N_RND - 1],
           half=TOP).wait_recv()
        mk(o8, left, send_l.at[N_RND - 1], recv_l.at[N_RND - 1],
           half=BOT).wait_recv()
        chunk_out(o8)

        for s in sends:
            s.wait_send()

    return pl.pallas_call(
        body,
        out_shape=jax.ShapeDtypeStruct((N_DEV * m_per, n_per), jnp.float32),
        in_specs=[
            pl.BlockSpec(memory_space=pltpu.VMEM),
            pl.BlockSpec(memory_space=pltpu.VMEM),
            pl.BlockSpec(memory_space=pltpu.SMEM),
            pl.BlockSpec(memory_space=pltpu.SMEM),
        ],
        out_specs=pl.BlockSpec(memory_space=pltpu.VMEM),
        scratch_shapes=[
            pltpu.VMEM((N_DEV, m_per, k), jnp.float8_e5m2),
            pltpu.VMEM((k, n_per), jnp.float8_e5m2),
            pltpu.SemaphoreType.DMA((N_RND,)),
            pltpu.SemaphoreType.DMA((N_RND,)),
            pltpu.SemaphoreType.DMA((N_RND,)),
            pltpu.SemaphoreType.DMA((N_RND,)),
        ],
        compiler_params=pltpu.CompilerParams(collective_id=0),
    )(x, w_mat, scale_x, scale_w)


# device time: 109997 ns/iter; 1.9252x vs baseline; 1.0025x over previous
import jax
import jax.numpy as jnp
from jax import lax
from jax.experimental import pallas as pl
from jax.experimental.pallas import tpu as pltpu

N_DEV = 16
N_RND = 8
TOP, BOT = 0, 1


def kernel(x, w_mat, scale_x, scale_w):
    m_per, k = x.shape
    _, n_per = w_mat.shape
    m_half = m_per // 2

    def body(x_ref, w_ref, sx_ref, sw_ref, out_ref, xg_ref, w8_ref,
             send_r, send_l, recv_r, recv_l):
        my = lax.axis_index("i")
        left = lax.rem(my + N_DEV - 1, N_DEV)
        right = lax.rem(my + 1, N_DEV)

        xg_ref[my] = x_ref[...].astype(jnp.float8_e5m2)

        barrier_sem = pltpu.get_barrier_semaphore()
        pl.semaphore_signal(barrier_sem, inc=1, device_id=(left,),
                            device_id_type=pl.DeviceIdType.MESH)
        pl.semaphore_signal(barrier_sem, inc=1, device_id=(right,),
                            device_id_type=pl.DeviceIdType.MESH)
        pl.semaphore_wait(barrier_sem, 2)

        scale = sx_ref[0] * sw_ref[0]

        def chunk_out(org):
            xc = xg_ref[org]
            acc = lax.dot_general(
                xc, w8_ref[...], (((1,), (0,)), ((), ())),
                preferred_element_type=jnp.float32)
            out_ref[pl.ds(org * m_per, m_per), :] = jnp.maximum(
                acc * scale, 0.0)

        def mk(org, dev, send_sem, recv_sem, half=None):
            if half is None:
                src = xg_ref.at[org]
            else:
                src = xg_ref.at[org, pl.ds(half * m_half, m_half)]
            return pltpu.make_async_remote_copy(
                src_ref=src, dst_ref=src,
                send_sem=send_sem, recv_sem=recv_sem,
                device_id=(dev,), device_id_type=pl.DeviceIdType.MESH,
            )

        sends = []

        def start(org, dev, send_sems, recv_sems, r, half=None):
            s = mk(org, dev, send_sems.at[r], recv_sems.at[r], half)
            s.start()
            sends.append(s)

        start(my, right, send_r, recv_r, 0)
        start(my, left, send_l, recv_l, 0)
        w8_ref[...] = w_ref[...].astype(jnp.float8_e5m2)
        chunk_out(my)

        for r in range(N_RND - 1):
            o_r = lax.rem(my - 1 - r + N_DEV, N_DEV)
            o_l = lax.rem(my + 1 + r, N_DEV)
            mk(o_r, right, send_r.at[r], recv_r.at[r]).wait_recv()
            if r < N_RND - 2:
                start(o_r, right, send_r, recv_r, r + 1)
            else:
                start(o_r, right, send_r, recv_r, r + 1, half=TOP)
            mk(o_l, left, send_l.at[r], recv_l.at[r]).wait_recv()
            if r < N_RND - 2:
                start(o_l, left, send_l, recv_l, r + 1)
            else:
                start(o_l, left, send_l, recv_l, r + 1, half=BOT)
            chunk_out(o_r)
            chunk_out(o_l)

        o8 = lax.rem(my + N_DEV // 2, N_DEV)
        mk(o8, right, send_r.at[N_RND - 1], recv_r.at[N_RND - 1],
           half=TOP).wait_recv()
        mk(o8, left, send_l.at[N_RND - 1], recv_l.at[N_RND - 1],
           half=BOT).wait_recv()
        chunk_out(o8)

        for s in sends:
            s.wait_send()

    return pl.pallas_call(
        body,
        out_shape=jax.ShapeDtypeStruct((N_DEV * m_per, n_per), jnp.float32),
        in_specs=[
            pl.BlockSpec(memory_space=pltpu.VMEM),
            pl.BlockSpec(memory_space=pltpu.VMEM),
            pl.BlockSpec(memory_space=pltpu.SMEM),
            pl.BlockSpec(memory_space=pltpu.SMEM),
        ],
        out_specs=pl.BlockSpec(memory_space=pltpu.VMEM),
        scratch_shapes=[
            pltpu.VMEM((N_DEV, m_per, k), jnp.float8_e5m2),
            pltpu.VMEM((k, n_per), jnp.float8_e5m2),
            pltpu.SemaphoreType.DMA((N_RND,)),
            pltpu.SemaphoreType.DMA((N_RND,)),
            pltpu.SemaphoreType.DMA((N_RND,)),
            pltpu.SemaphoreType.DMA((N_RND,)),
        ],
        compiler_params=pltpu.CompilerParams(collective_id=0),
    )(x, w_mat, scale_x, scale_w)
